# baseline (speedup 1.0000x reference)
import functools

import jax
import jax.numpy as jnp
from jax import lax
from jax.experimental import pallas as pl
from jax.experimental.pallas import tpu as pltpu
from jax.experimental.pallas import tpu_sc as plsc

N = 10000
E = 320000
D = 128

NC = 2
NS = 16
L = 16
NW = NC * NS

_MESH = plsc.VectorSubcoreMesh(core_axis_name="c", subcore_axis_name="s")

BD = 80
JD = E // (NW * BD)

BA = 80
JA = E // (NW * BA)
NJ = N

_ZCH = 400
_NZC = N // _ZCH
_RCH = 80
_NRC = N // _RCH


def _fill1d(ref, n, value):
    def body(i, _):
        ref[pl.ds(i * L, L)] = jnp.full((L,), value, jnp.float32)
        return 0
    lax.fori_loop(0, n // L, body, 0)


@functools.partial(
    pl.kernel,
    out_type=jax.ShapeDtypeStruct((NC * N,), jnp.float32),
    mesh=_MESH,
    scratch_types=[
        pltpu.VMEM((3, BD), jnp.int32),
        pltpu.VMEM((BD,), jnp.float32),
        pltpu.VMEM((_ZCH,), jnp.float32),
        pltpu.VMEM_SHARED((N,), jnp.float32),
        pltpu.SemaphoreType.DMA((3,)),
        pltpu.SemaphoreType.DMA((3,)),
    ],
)
def _sc_src_degrees(src_hbm, out_hbm, idx_v, ones_v, zbuf_v, hist_sh,
                    lsem, hsem):
    c = lax.axis_index("c")
    s = lax.axis_index("s")
    base = (c * NS + s) * JD * BD

    _fill1d(ones_v, BD, 1.0)
    _fill1d(zbuf_v, _ZCH, 0.0)

    def zero_chunk(j, _):
        ch = s + NS * j

        @pl.when(ch < _NZC)
        def _():
            pltpu.sync_copy(zbuf_v, hist_sh.at[pl.ds(ch * _ZCH, _ZCH)])

        return 0

    lax.fori_loop(0, (_NZC + NS - 1) // NS, zero_chunk, 0)
    plsc.subcore_barrier()

    def load(j):
        b = j % 3
        return pltpu.make_async_copy(
            src_hbm.at[pl.ds(base + j * BD, BD)], idx_v.at[b], lsem.at[b])

    def hscat_wait(j):
        b = j % 3
        pltpu.make_async_copy(ones_v, hist_sh.at[idx_v.at[b]],
                              hsem.at[b]).wait()

    load(0).start()
    load(1).start()

    def burst(j, _):
        b = j % 3
        load(j).wait()

        @pl.when(j >= 1)
        def _():
            hscat_wait(j - 1)

        @pl.when(j + 2 < JD)
        def _():
            load(j + 2).start()

        pltpu.async_copy(ones_v, hist_sh.at[idx_v.at[b]], hsem.at[b],
                         add=True)
        return 0

    lax.fori_loop(0, JD, burst, 0)
    hscat_wait(JD - 1)
    plsc.subcore_barrier()

    def write_chunk(j, _):
        ch = s + NS * j

        @pl.when(ch < _NZC)
        def _():
            pltpu.sync_copy(hist_sh.at[pl.ds(ch * _ZCH, _ZCH)], zbuf_v)
            pltpu.sync_copy(zbuf_v,
                            out_hbm.at[pl.ds(c * N + ch * _ZCH, _ZCH)])

        return 0

    lax.fori_loop(0, (_NZC + NS - 1) // NS, write_chunk, 0)


@functools.partial(
    pl.kernel,
    out_type=(
        jax.ShapeDtypeStruct((NC, N, D), jnp.float32),
        jax.ShapeDtypeStruct((NC * N,), jnp.float32),
    ),
    mesh=_MESH,
    scratch_types=[
        pltpu.VMEM((4, BA), jnp.int32),
        pltpu.VMEM((4, BA), jnp.int32),
        pltpu.VMEM((4, BA, D), jnp.float32),
        pltpu.VMEM((BA,), jnp.float32),
        pltpu.VMEM((_ZCH,), jnp.float32),
        pltpu.VMEM_SHARED((NJ, D), jnp.float32),
        pltpu.VMEM_SHARED((NJ,), jnp.float32),
        pltpu.SemaphoreType.DMA((4,)),
        pltpu.SemaphoreType.DMA((4,)),
        pltpu.SemaphoreType.DMA((4,)),
        pltpu.SemaphoreType.DMA((4,)),
        pltpu.SemaphoreType.DMA((4,)),
        pltpu.SemaphoreType.DMA,
    ],
)
def _sc_aggregate(h_hbm, edge_hbm, out_hbm, hout_hbm,
                  sidx_v, didx_v, rows_v, ones_v, zbuf_v,
                  agg_sh, hist_sh, ssem, dsem, gsem, asem, hsem, xsem):
    c = lax.axis_index("c")
    s = lax.axis_index("s")
    base = (c * NS + s) * JA * BA

    _fill1d(ones_v, BA, 1.0)
    _fill1d(zbuf_v, _ZCH, 0.0)

    def fill_zero(k, _):
        rows_v[0, k // (D // L), pl.ds((k % (D // L)) * L, L)] = (
            jnp.zeros((L,), jnp.float32))
        return 0

    lax.fori_loop(0, _RCH * (D // L), fill_zero, 0)

    def zero_cp(ch):
        return pltpu.make_async_copy(
            rows_v.at[0, pl.ds(0, _RCH)],
            agg_sh.at[pl.ds(ch * _RCH, _RCH)], xsem)

    def zero_chunk(j, _):
        ch = s + NS * j

        @pl.when(ch < _NRC)
        def _():
            zero_cp(ch).start()

        @pl.when(ch < _NZC)
        def _():
            pltpu.sync_copy(zbuf_v, hist_sh.at[pl.ds(ch * _ZCH, _ZCH)])

        return 0

    def zero_drain(j, _):
        ch = s + NS * j

        @pl.when(ch < _NRC)
        def _():
            zero_cp(ch).wait()

        return 0

    lax.fori_loop(0, (_NRC + NS - 1) // NS, zero_chunk, 0)
    lax.fori_loop(0, (_NRC + NS - 1) // NS, zero_drain, 0)
    plsc.subcore_barrier()

    def loads(j):
        b = j % 4
        return (
            pltpu.make_async_copy(
                edge_hbm.at[pl.ds(base + j * BA, BA)], sidx_v.at[b],
                ssem.at[b]),
            pltpu.make_async_copy(
                edge_hbm.at[pl.ds(E + base + j * BA, BA)], didx_v.at[b],
                dsem.at[b]),
        )

    def gather(j):
        b = j % 4
        return pltpu.make_async_copy(
            h_hbm.at[sidx_v.at[b]], rows_v.at[b], gsem.at[b])

    def scats_start(j):
        b = j % 4
        pltpu.async_copy(rows_v.at[b], agg_sh.at[didx_v.at[b]],
                         asem.at[b], add=True)
        pltpu.async_copy(ones_v, hist_sh.at[didx_v.at[b]],
                         hsem.at[b], add=True)

    def scats_wait(j):
        b = j % 4
        pltpu.make_async_copy(rows_v.at[b], agg_sh.at[didx_v.at[b]],
                              asem.at[b]).wait()
        pltpu.make_async_copy(ones_v, hist_sh.at[didx_v.at[b]],
                              hsem.at[b]).wait()

    for cp in loads(0):
        cp.start()
    for cp in loads(1):
        cp.start()
    for cp in loads(0):
        cp.wait()
    gather(0).start()

    def burst(j, _):
        @pl.when(j >= 2)
        def _():
            scats_wait(j - 2)

        @pl.when(j + 2 < JA)
        def _():
            for cp in loads(j + 2):
                cp.start()

        gather(j).wait()

        @pl.when(j + 1 < JA)
        def _():
            for cp in loads(j + 1):
                cp.wait()
            gather(j + 1).start()

        scats_start(j)
        return 0

    lax.fori_loop(0, JA, burst, 0)
    scats_wait(JA - 2)
    scats_wait(JA - 1)
    plsc.subcore_barrier()

    def write_chunk(j, _):
        ch = s + NS * j

        @pl.when(ch < _NRC)
        def _():
            pltpu.sync_copy(agg_sh.at[pl.ds(ch * _RCH, _RCH)],
                            rows_v.at[0, pl.ds(0, _RCH)])
            pltpu.sync_copy(rows_v.at[0, pl.ds(0, _RCH)],
                            out_hbm.at[c, pl.ds(ch * _RCH, _RCH)])

        @pl.when(ch < _NZC)
        def _():
            pltpu.sync_copy(hist_sh.at[pl.ds(ch * _ZCH, _ZCH)], zbuf_v)
            pltpu.sync_copy(zbuf_v,
                            hout_hbm.at[pl.ds(c * N + ch * _ZCH, _ZCH)])

        return 0

    lax.fori_loop(0, (_NRC + NS - 1) // NS, write_chunk, 0)


_BM = 200


def _tc_scale_mm_body(f_ref, d_ref, w_ref, o_ref):
    deg = d_ref[0] + d_ref[1]
    norm = lax.rsqrt(jnp.maximum(deg, 1.0))
    h = f_ref[...] * norm
    o_ref[...] = jnp.dot(h, w_ref[...], preferred_element_type=jnp.float32)


def _tc_finish_body(p_ref, d_ref, b_ref, o_ref):
    agg = p_ref[0] + p_ref[1]
    deg = d_ref[0] + d_ref[1]
    norm = lax.rsqrt(jnp.maximum(deg, 1.0))
    o_ref[...] = jnp.maximum(agg * norm + b_ref[...], 0.0)


_tc_scale_mm = pl.pallas_call(
    _tc_scale_mm_body,
    grid=(N // _BM,),
    in_specs=[
        pl.BlockSpec((_BM, D), lambda i: (i, 0)),
        pl.BlockSpec((NC, _BM, 1), lambda i: (0, i, 0)),
        pl.BlockSpec((D, D), lambda i: (0, 0)),
    ],
    out_specs=pl.BlockSpec((_BM, D), lambda i: (i, 0)),
    out_shape=jax.ShapeDtypeStruct((N, D), jnp.float32),
)

_tc_finish = pl.pallas_call(
    _tc_finish_body,
    grid=(N // _BM,),
    in_specs=[
        pl.BlockSpec((NC, _BM, D), lambda i: (0, i, 0)),
        pl.BlockSpec((NC, _BM, 1), lambda i: (0, i, 0)),
        pl.BlockSpec((1, D), lambda i: (0, 0)),
    ],
    out_specs=pl.BlockSpec((_BM, D), lambda i: (i, 0)),
    out_shape=jax.ShapeDtypeStruct((N, D), jnp.float32),
)


@jax.jit
def kernel(feat, edge_index, W, b):
    eflat = edge_index.astype(jnp.int32).reshape(2 * E)
    degs = _sc_src_degrees(eflat).reshape(NC, N, 1)
    h = _tc_scale_mm(feat, degs, W)
    partials, hist = _sc_aggregate(h, eflat)
    return _tc_finish(partials, hist.reshape(NC, N, 1), b.reshape(1, D))


# speedup vs baseline: 1.2084x; 1.2084x over previous
import functools

import jax
import jax.numpy as jnp
from jax import lax
from jax.experimental import pallas as pl
from jax.experimental.pallas import tpu as pltpu
from jax.experimental.pallas import tpu_sc as plsc

N = 10000
E = 320000
D = 128

NC = 2
NS = 16
L = 16
NW = NC * NS

_MESH = plsc.VectorSubcoreMesh(core_axis_name="c", subcore_axis_name="s")

BD = 128
NBK = E // BD
NT = (NBK + NW - 1) // NW
NRE = NBK - (NT - 1) * NW

BA = 80
JA = E // (NW * BA)

_ZCH = 400
_NZC = N // _ZCH
_RCH = 80
_NRC = N // _RCH


def _fill1d(ref, n, value):
    def body(i, _):
        ref[pl.ds(i * L, L)] = jnp.full((L,), value, jnp.float32)
        return 0
    lax.fori_loop(0, n // L, body, 0)


@functools.partial(
    pl.kernel,
    out_type=(
        jax.ShapeDtypeStruct((NC * N,), jnp.float32),
        jax.ShapeDtypeStruct((2 * E,), jnp.int32),
    ),
    mesh=_MESH,
    scratch_types=[
        pltpu.VMEM((3, 2, BD), jnp.int32),
        pltpu.VMEM((BD,), jnp.float32),
        pltpu.VMEM((_ZCH,), jnp.float32),
        pltpu.VMEM_SHARED((N,), jnp.float32),
        pltpu.SemaphoreType.DMA((3,)),
        pltpu.SemaphoreType.DMA((3,)),
        pltpu.SemaphoreType.DMA((3,)),
        pltpu.SemaphoreType.DMA((3,)),
    ],
)
def _sc_src_degrees(ei_hbm, dp_hbm, ef_hbm,
                    eb_v, ones_v, zbuf_v, hist_sh, lsem, csem, w0sem, w1sem):
    c = lax.axis_index("c")
    s = lax.axis_index("s")
    w = c * NS + s
    nv = NT - 1 + jnp.where(w < NRE, 1, 0)

    _fill1d(ones_v, BD, 1.0)
    _fill1d(zbuf_v, _ZCH, 0.0)

    def zero_chunk(j, _):
        ch = s + NS * j

        @pl.when(ch < _NZC)
        def _():
            pltpu.sync_copy(zbuf_v, hist_sh.at[pl.ds(ch * _ZCH, _ZCH)])

        return 0

    lax.fori_loop(0, (_NZC + NS - 1) // NS, zero_chunk, 0)
    plsc.subcore_barrier()

    def off(t):
        return (w + NW * t) * BD

    def load(t, b):
        return pltpu.make_async_copy(
            ei_hbm.at[:, pl.ds(off(t), BD)], eb_v.at[b], lsem.at[b])

    def work_start(t, b):
        pltpu.async_copy(ones_v, hist_sh.at[eb_v.at[b, 0]], csem.at[b],
                         add=True)
        pltpu.async_copy(eb_v.at[b, 0], ef_hbm.at[pl.ds(off(t), BD)],
                         w0sem.at[b])
        pltpu.async_copy(eb_v.at[b, 1], ef_hbm.at[pl.ds(E + off(t), BD)],
                         w1sem.at[b])

    def work_wait(t, b):
        pltpu.make_async_copy(ones_v, hist_sh.at[eb_v.at[b, 0]],
                              csem.at[b]).wait()
        pltpu.make_async_copy(eb_v.at[b, 0], ef_hbm.at[pl.ds(off(t), BD)],
                              w0sem.at[b]).wait()
        pltpu.make_async_copy(eb_v.at[b, 1],
                              ef_hbm.at[pl.ds(E + off(t), BD)],
                              w1sem.at[b]).wait()

    load(0, 0).start()
    load(1, 1).start()

    def step(to, _):
        for tb in range(3):
            t = 3 * to + tb

            @pl.when(t < nv)
            def _():
                load(t, tb).wait()
                work_start(t, tb)

            @pl.when((t >= 1) & (t - 1 < nv))
            def _():
                work_wait(t - 1, (tb - 1) % 3)

            @pl.when(t + 2 < nv)
            def _():
                load(t + 2, (tb + 2) % 3).start()

        return 0

    lax.fori_loop(0, NT // 3 + 1, step, 0)
    plsc.subcore_barrier()

    def write_chunk(j, _):
        ch = s + NS * j

        @pl.when(ch < _NZC)
        def _():
            pltpu.sync_copy(hist_sh.at[pl.ds(ch * _ZCH, _ZCH)], zbuf_v)
            pltpu.sync_copy(zbuf_v,
                            dp_hbm.at[pl.ds(c * N + ch * _ZCH, _ZCH)])

        return 0

    lax.fori_loop(0, (_NZC + NS - 1) // NS, write_chunk, 0)


@functools.partial(
    pl.kernel,
    out_type=(
        jax.ShapeDtypeStruct((NC, N, D), jnp.float32),
        jax.ShapeDtypeStruct((NC * N,), jnp.float32),
    ),
    mesh=_MESH,
    scratch_types=[
        pltpu.VMEM((4, BA), jnp.int32),
        pltpu.VMEM((4, BA), jnp.int32),
        pltpu.VMEM((4, BA, D), jnp.float32),
        pltpu.VMEM((BA,), jnp.float32),
        pltpu.VMEM((_ZCH,), jnp.float32),
        pltpu.VMEM_SHARED((N, D), jnp.float32),
        pltpu.VMEM_SHARED((N,), jnp.float32),
        pltpu.SemaphoreType.DMA((4,)),
        pltpu.SemaphoreType.DMA((4,)),
        pltpu.SemaphoreType.DMA((4,)),
        pltpu.SemaphoreType.DMA((4,)),
        pltpu.SemaphoreType.DMA((4,)),
        pltpu.SemaphoreType.DMA,
    ],
)
def _sc_aggregate(h_hbm, edge_hbm, out_hbm, hout_hbm,
                  sidx_v, didx_v, rows_v, ones_v, zbuf_v,
                  agg_sh, hist_sh, ssem, dsem, gsem, asem, hsem, xsem):
    c = lax.axis_index("c")
    s = lax.axis_index("s")
    base = (c * NS + s) * JA * BA

    _fill1d(ones_v, BA, 1.0)
    _fill1d(zbuf_v, _ZCH, 0.0)

    def fill_zero(k, _):
        rows_v[0, k // (D // L), pl.ds((k % (D // L)) * L, L)] = (
            jnp.zeros((L,), jnp.float32))
        return 0

    lax.fori_loop(0, _RCH * (D // L), fill_zero, 0)

    def zero_cp(ch):
        return pltpu.make_async_copy(
            rows_v.at[0, pl.ds(0, _RCH)],
            agg_sh.at[pl.ds(ch * _RCH, _RCH)], xsem)

    def zero_chunk(j, _):
        ch = s + NS * j

        @pl.when(ch < _NRC)
        def _():
            zero_cp(ch).start()

        @pl.when(ch < _NZC)
        def _():
            pltpu.sync_copy(zbuf_v, hist_sh.at[pl.ds(ch * _ZCH, _ZCH)])

        return 0

    def zero_drain(j, _):
        ch = s + NS * j

        @pl.when(ch < _NRC)
        def _():
            zero_cp(ch).wait()

        return 0

    lax.fori_loop(0, (_NRC + NS - 1) // NS, zero_chunk, 0)
    lax.fori_loop(0, (_NRC + NS - 1) // NS, zero_drain, 0)
    plsc.subcore_barrier()

    def loads(j):
        b = j % 4
        return (
            pltpu.make_async_copy(
                edge_hbm.at[pl.ds(base + j * BA, BA)], sidx_v.at[b],
                ssem.at[b]),
            pltpu.make_async_copy(
                edge_hbm.at[pl.ds(E + base + j * BA, BA)], didx_v.at[b],
                dsem.at[b]),
        )

    def gather(j):
        b = j % 4
        return pltpu.make_async_copy(
            h_hbm.at[sidx_v.at[b]], rows_v.at[b], gsem.at[b])

    def scats_start(j):
        b = j % 4
        pltpu.async_copy(rows_v.at[b], agg_sh.at[didx_v.at[b]],
                         asem.at[b], add=True)
        pltpu.async_copy(ones_v, hist_sh.at[didx_v.at[b]],
                         hsem.at[b], add=True)

    def scats_wait(j):
        b = j % 4
        pltpu.make_async_copy(rows_v.at[b], agg_sh.at[didx_v.at[b]],
                              asem.at[b]).wait()
        pltpu.make_async_copy(ones_v, hist_sh.at[didx_v.at[b]],
                              hsem.at[b]).wait()

    for cp in loads(0):
        cp.start()
    for cp in loads(1):
        cp.start()
    for cp in loads(0):
        cp.wait()
    gather(0).start()

    def burst(j, _):
        @pl.when(j >= 2)
        def _():
            scats_wait(j - 2)

        @pl.when(j + 2 < JA)
        def _():
            for cp in loads(j + 2):
                cp.start()

        gather(j).wait()

        @pl.when(j + 1 < JA)
        def _():
            for cp in loads(j + 1):
                cp.wait()
            gather(j + 1).start()

        scats_start(j)
        return 0

    lax.fori_loop(0, JA, burst, 0)
    scats_wait(JA - 2)
    scats_wait(JA - 1)
    plsc.subcore_barrier()

    def write_chunk(j, _):
        ch = s + NS * j

        @pl.when(ch < _NRC)
        def _():
            pltpu.sync_copy(agg_sh.at[pl.ds(ch * _RCH, _RCH)],
                            rows_v.at[0, pl.ds(0, _RCH)])
            pltpu.sync_copy(rows_v.at[0, pl.ds(0, _RCH)],
                            out_hbm.at[c, pl.ds(ch * _RCH, _RCH)])

        @pl.when(ch < _NZC)
        def _():
            pltpu.sync_copy(hist_sh.at[pl.ds(ch * _ZCH, _ZCH)], zbuf_v)
            pltpu.sync_copy(zbuf_v,
                            hout_hbm.at[pl.ds(c * N + ch * _ZCH, _ZCH)])

        return 0

    lax.fori_loop(0, (_NRC + NS - 1) // NS, write_chunk, 0)


_BM = 1000


def _tc_scale_mm_body(f_ref, d_ref, w_ref, o_ref):
    deg = d_ref[0] + d_ref[1]
    norm = lax.rsqrt(jnp.maximum(deg, 1.0))
    h = f_ref[...] * norm
    o_ref[...] = jnp.dot(h, w_ref[...], preferred_element_type=jnp.float32)


def _tc_finish_body(p_ref, d_ref, b_ref, o_ref):
    agg = p_ref[0] + p_ref[1]
    deg = d_ref[0] + d_ref[1]
    norm = lax.rsqrt(jnp.maximum(deg, 1.0))
    o_ref[...] = jnp.maximum(agg * norm + b_ref[...], 0.0)


_tc_scale_mm = pl.pallas_call(
    _tc_scale_mm_body,
    grid=(N // _BM,),
    in_specs=[
        pl.BlockSpec((_BM, D), lambda i: (i, 0)),
        pl.BlockSpec((NC, _BM, 1), lambda i: (0, i, 0)),
        pl.BlockSpec((D, D), lambda i: (0, 0)),
    ],
    out_specs=pl.BlockSpec((_BM, D), lambda i: (i, 0)),
    out_shape=jax.ShapeDtypeStruct((N, D), jnp.float32),
)

_tc_finish = pl.pallas_call(
    _tc_finish_body,
    grid=(N // _BM,),
    in_specs=[
        pl.BlockSpec((NC, _BM, D), lambda i: (0, i, 0)),
        pl.BlockSpec((NC, _BM, 1), lambda i: (0, i, 0)),
        pl.BlockSpec((1, D), lambda i: (0, 0)),
    ],
    out_specs=pl.BlockSpec((_BM, D), lambda i: (i, 0)),
    out_shape=jax.ShapeDtypeStruct((N, D), jnp.float32),
)


@jax.jit
def kernel(feat, edge_index, W, b):
    ei = edge_index.astype(jnp.int32)
    degs, eflat = _sc_src_degrees(ei)
    h = _tc_scale_mm(feat, degs.reshape(NC, N, 1), W)
    partials, hist = _sc_aggregate(h, eflat)
    return _tc_finish(partials, hist.reshape(NC, N, 1), b.reshape(1, D))


# speedup vs baseline: 1.2773x; 1.0570x over previous
import functools

import jax
import jax.numpy as jnp
from jax import lax
from jax.experimental import pallas as pl
from jax.experimental.pallas import tpu as pltpu
from jax.experimental.pallas import tpu_sc as plsc

N = 10000
E = 320000
D = 128

NC = 2
NS = 16
L = 16
NW = NC * NS

_MESH = plsc.VectorSubcoreMesh(core_axis_name="c", subcore_axis_name="s")

BD = 128
NBK = E // BD
NT = (NBK + NW - 1) // NW
NRE = NBK - (NT - 1) * NW

BA = 80
JA = E // (NW * BA)

_ZCH = 200
_NZC = N // _ZCH
_BM = 1000
_DST = 2048
_DSZ = (N // _BM) * _DST
_RCH = 80
_NRC = N // _RCH


def _fill1d(ref, n, value):
    def body(i, _):
        ref[pl.ds(i * L, L)] = jnp.full((L,), value, jnp.float32)
        return 0
    lax.fori_loop(0, n // L, body, 0)


@functools.partial(
    pl.kernel,
    out_type=(
        jax.ShapeDtypeStruct((_DSZ,), jnp.float32),
        jax.ShapeDtypeStruct((2 * E,), jnp.int32),
    ),
    mesh=_MESH,
    scratch_types=[
        pltpu.VMEM((3, 2, BD), jnp.int32),
        pltpu.VMEM((BD,), jnp.float32),
        pltpu.VMEM((208,), jnp.float32),
        pltpu.VMEM_SHARED((N,), jnp.float32),
        pltpu.SemaphoreType.DMA((3,)),
        pltpu.SemaphoreType.DMA((3,)),
        pltpu.SemaphoreType.DMA((3,)),
        pltpu.SemaphoreType.DMA((3,)),
    ],
)
def _sc_src_degrees(ei_hbm, dp_hbm, ef_hbm,
                    eb_v, ones_v, zbuf_v, hist_sh, lsem, csem, w0sem, w1sem):
    c = lax.axis_index("c")
    s = lax.axis_index("s")
    w = c * NS + s
    nv = NT - 1 + jnp.where(w < NRE, 1, 0)

    _fill1d(ones_v, BD, 1.0)
    _fill1d(zbuf_v, 208, 0.0)

    def zero_chunk(j, _):
        ch = s + NS * j

        @pl.when(ch < _NZC)
        def _():
            pltpu.sync_copy(zbuf_v.at[pl.ds(0, _ZCH)],
                            hist_sh.at[pl.ds(ch * _ZCH, _ZCH)])

        return 0

    lax.fori_loop(0, (_NZC + NS - 1) // NS, zero_chunk, 0)
    plsc.subcore_barrier()

    def off(t):
        return (w + NW * t) * BD

    def load(t, b):
        return pltpu.make_async_copy(
            ei_hbm.at[:, pl.ds(off(t), BD)], eb_v.at[b], lsem.at[b])

    def work_start(t, b):
        pltpu.async_copy(ones_v, hist_sh.at[eb_v.at[b, 0]], csem.at[b],
                         add=True)
        pltpu.async_copy(eb_v.at[b, 0], ef_hbm.at[pl.ds(off(t), BD)],
                         w0sem.at[b])
        pltpu.async_copy(eb_v.at[b, 1], ef_hbm.at[pl.ds(E + off(t), BD)],
                         w1sem.at[b])

    def work_wait(t, b):
        pltpu.make_async_copy(ones_v, hist_sh.at[eb_v.at[b, 0]],
                              csem.at[b]).wait()
        pltpu.make_async_copy(eb_v.at[b, 0], ef_hbm.at[pl.ds(off(t), BD)],
                              w0sem.at[b]).wait()
        pltpu.make_async_copy(eb_v.at[b, 1],
                              ef_hbm.at[pl.ds(E + off(t), BD)],
                              w1sem.at[b]).wait()

    load(0, 0).start()
    load(1, 1).start()

    def step(to, _):
        for tb in range(3):
            t = 3 * to + tb

            @pl.when(t < nv)
            def _():
                load(t, tb).wait()
                work_start(t, tb)

            @pl.when((t >= 1) & (t - 1 < nv))
            def _():
                work_wait(t - 1, (tb - 1) % 3)

            @pl.when(t + 2 < nv)
            def _():
                load(t + 2, (tb + 2) % 3).start()

        return 0

    lax.fori_loop(0, NT // 3 + 1, step, 0)
    plsc.subcore_barrier()

    def write_chunk(j, _):
        ch = s + NS * j

        @pl.when(ch < _NZC)
        def _():
            dest = (ch // 5) * _DST + c * 1024 + (ch % 5) * _ZCH
            pltpu.sync_copy(hist_sh.at[pl.ds(ch * _ZCH, _ZCH)],
                            zbuf_v.at[pl.ds(0, _ZCH)])
            pltpu.sync_copy(zbuf_v.at[pl.ds(0, _ZCH)],
                            dp_hbm.at[pl.ds(dest, _ZCH)])

        return 0

    lax.fori_loop(0, (_NZC + NS - 1) // NS, write_chunk, 0)


@functools.partial(
    pl.kernel,
    out_type=(
        jax.ShapeDtypeStruct((NC, N, D), jnp.float32),
        jax.ShapeDtypeStruct((_DSZ,), jnp.float32),
    ),
    mesh=_MESH,
    scratch_types=[
        pltpu.VMEM((4, BA), jnp.int32),
        pltpu.VMEM((4, BA), jnp.int32),
        pltpu.VMEM((4, BA, D), jnp.float32),
        pltpu.VMEM((BA,), jnp.float32),
        pltpu.VMEM((208,), jnp.float32),
        pltpu.VMEM_SHARED((N, D), jnp.float32),
        pltpu.VMEM_SHARED((N,), jnp.float32),
        pltpu.SemaphoreType.DMA((4,)),
        pltpu.SemaphoreType.DMA((4,)),
        pltpu.SemaphoreType.DMA((4,)),
        pltpu.SemaphoreType.DMA((4,)),
        pltpu.SemaphoreType.DMA((4,)),
        pltpu.SemaphoreType.DMA,
    ],
)
def _sc_aggregate(h_hbm, edge_hbm, out_hbm, hout_hbm,
                  sidx_v, didx_v, rows_v, ones_v, zbuf_v,
                  agg_sh, hist_sh, ssem, dsem, gsem, asem, hsem, xsem):
    c = lax.axis_index("c")
    s = lax.axis_index("s")
    base = (c * NS + s) * JA * BA

    _fill1d(ones_v, BA, 1.0)
    _fill1d(zbuf_v, 208, 0.0)

    def fill_zero(k, _):
        rows_v[0, k // (D // L), pl.ds((k % (D // L)) * L, L)] = (
            jnp.zeros((L,), jnp.float32))
        return 0

    lax.fori_loop(0, _RCH * (D // L), fill_zero, 0)

    def zero_cp(ch):
        return pltpu.make_async_copy(
            rows_v.at[0, pl.ds(0, _RCH)],
            agg_sh.at[pl.ds(ch * _RCH, _RCH)], xsem)

    def zero_chunk(j, _):
        ch = s + NS * j

        @pl.when(ch < _NRC)
        def _():
            zero_cp(ch).start()

        @pl.when(ch < _NZC)
        def _():
            pltpu.sync_copy(zbuf_v.at[pl.ds(0, _ZCH)],
                            hist_sh.at[pl.ds(ch * _ZCH, _ZCH)])

        return 0

    def zero_drain(j, _):
        ch = s + NS * j

        @pl.when(ch < _NRC)
        def _():
            zero_cp(ch).wait()

        return 0

    lax.fori_loop(0, (_NRC + NS - 1) // NS, zero_chunk, 0)
    lax.fori_loop(0, (_NRC + NS - 1) // NS, zero_drain, 0)
    plsc.subcore_barrier()

    def loads(j):
        b = j % 4
        return (
            pltpu.make_async_copy(
                edge_hbm.at[pl.ds(base + j * BA, BA)], sidx_v.at[b],
                ssem.at[b]),
            pltpu.make_async_copy(
                edge_hbm.at[pl.ds(E + base + j * BA, BA)], didx_v.at[b],
                dsem.at[b]),
        )

    def gather(j):
        b = j % 4
        return pltpu.make_async_copy(
            h_hbm.at[sidx_v.at[b]], rows_v.at[b], gsem.at[b])

    def scats_start(j):
        b = j % 4
        pltpu.async_copy(rows_v.at[b], agg_sh.at[didx_v.at[b]],
                         asem.at[b], add=True)
        pltpu.async_copy(ones_v, hist_sh.at[didx_v.at[b]],
                         hsem.at[b], add=True)

    def scats_wait(j):
        b = j % 4
        pltpu.make_async_copy(rows_v.at[b], agg_sh.at[didx_v.at[b]],
                              asem.at[b]).wait()
        pltpu.make_async_copy(ones_v, hist_sh.at[didx_v.at[b]],
                              hsem.at[b]).wait()

    for cp in loads(0):
        cp.start()
    for cp in loads(1):
        cp.start()
    for cp in loads(0):
        cp.wait()
    gather(0).start()

    def burst(j, _):
        @pl.when(j >= 2)
        def _():
            scats_wait(j - 2)

        @pl.when(j + 2 < JA)
        def _():
            for cp in loads(j + 2):
                cp.start()

        gather(j).wait()

        @pl.when(j + 1 < JA)
        def _():
            for cp in loads(j + 1):
                cp.wait()
            gather(j + 1).start()

        scats_start(j)
        return 0

    lax.fori_loop(0, JA, burst, 0)
    scats_wait(JA - 2)
    scats_wait(JA - 1)
    plsc.subcore_barrier()

    def write_chunk(j, _):
        ch = s + NS * j

        @pl.when(ch < _NRC)
        def _():
            pltpu.sync_copy(agg_sh.at[pl.ds(ch * _RCH, _RCH)],
                            rows_v.at[0, pl.ds(0, _RCH)])
            pltpu.sync_copy(rows_v.at[0, pl.ds(0, _RCH)],
                            out_hbm.at[c, pl.ds(ch * _RCH, _RCH)])

        @pl.when(ch < _NZC)
        def _():
            dest = (ch // 5) * _DST + c * 1024 + (ch % 5) * _ZCH
            pltpu.sync_copy(hist_sh.at[pl.ds(ch * _ZCH, _ZCH)],
                            zbuf_v.at[pl.ds(0, _ZCH)])
            pltpu.sync_copy(zbuf_v.at[pl.ds(0, _ZCH)],
                            hout_hbm.at[pl.ds(dest, _ZCH)])

        return 0

    lax.fori_loop(0, (_NRC + NS - 1) // NS, write_chunk, 0)


def _norm_from_flat(d_hbm, dv0, dv1, s0, s1):
    i = pl.program_id(0)
    cp0 = pltpu.make_async_copy(d_hbm.at[pl.ds(i * _DST, 1024)], dv0, s0)
    cp1 = pltpu.make_async_copy(d_hbm.at[pl.ds(i * _DST + 1024, 1024)],
                                dv1, s1)
    cp0.start()
    cp1.start()
    cp0.wait()
    cp1.wait()
    deg = dv0[...] + dv1[...]
    norm = lax.rsqrt(jnp.maximum(deg, 1.0))
    return norm[0:_BM].reshape(_BM, 1)


def _tc_scale_mm_body(d_hbm, f_ref, w_ref, o_ref, dv0, dv1, s0, s1):
    norm = _norm_from_flat(d_hbm, dv0, dv1, s0, s1)
    h = f_ref[...] * norm
    o_ref[...] = jnp.dot(h, w_ref[...], preferred_element_type=jnp.float32)


def _tc_finish_body(d_hbm, p_ref, b_ref, o_ref, dv0, dv1, s0, s1):
    norm = _norm_from_flat(d_hbm, dv0, dv1, s0, s1)
    agg = p_ref[0] + p_ref[1]
    o_ref[...] = jnp.maximum(agg * norm + b_ref[...], 0.0)


_tc_scale_mm = pl.pallas_call(
    _tc_scale_mm_body,
    grid=(N // _BM,),
    in_specs=[
        pl.BlockSpec(memory_space=pl.ANY),
        pl.BlockSpec((_BM, D), lambda i: (i, 0)),
        pl.BlockSpec((D, D), lambda i: (0, 0)),
    ],
    out_specs=pl.BlockSpec((_BM, D), lambda i: (i, 0)),
    out_shape=jax.ShapeDtypeStruct((N, D), jnp.float32),
    scratch_shapes=[
        pltpu.VMEM((1024,), jnp.float32),
        pltpu.VMEM((1024,), jnp.float32),
        pltpu.SemaphoreType.DMA,
        pltpu.SemaphoreType.DMA,
    ],
)

_tc_finish = pl.pallas_call(
    _tc_finish_body,
    grid=(N // _BM,),
    in_specs=[
        pl.BlockSpec(memory_space=pl.ANY),
        pl.BlockSpec((NC, _BM, D), lambda i: (0, i, 0)),
        pl.BlockSpec((1, D), lambda i: (0, 0)),
    ],
    out_specs=pl.BlockSpec((_BM, D), lambda i: (i, 0)),
    out_shape=jax.ShapeDtypeStruct((N, D), jnp.float32),
    scratch_shapes=[
        pltpu.VMEM((1024,), jnp.float32),
        pltpu.VMEM((1024,), jnp.float32),
        pltpu.SemaphoreType.DMA,
        pltpu.SemaphoreType.DMA,
    ],
)


@jax.jit
def kernel(feat, edge_index, W, b):
    ei = edge_index.astype(jnp.int32)
    degs, eflat = _sc_src_degrees(ei)
    h = _tc_scale_mm(degs, feat, W)
    partials, hist = _sc_aggregate(h, eflat)
    return _tc_finish(hist, partials, b.reshape(1, D))
